# Initial kernel scaffold; baseline (speedup 1.0000x reference)
import jax
import jax.numpy as jnp
from jax.experimental import pallas as pl


def kernel(logits, ln_w, ln_b, W1, b1, W2, b2):
    raise NotImplementedError("write your pallas kernel here")


# speedup vs baseline: 2.5368x; 2.5368x over previous
import functools

import jax
import jax.numpy as jnp
from jax import lax
from jax.experimental import pallas as pl
from jax.experimental.pallas import tpu as pltpu
from jax.experimental.pallas import tpu_sc as plsc

B = 128
V = 100000
TOPK = 32
HID = 128
OUT = 128

NC = 2
NS = 16
NW = NC * NS
ROWS_PER_W = B // NW
LANES = 16
GROUP = 10
NVEC = V // LANES
NGROUP = NVEC // GROUP

_NEG = float("-inf")


def _sortd(x):
  return lax.rev(jnp.sort(x), (0,))


def _merge32(a, b, v):
  vs = _sortd(v)
  m = jnp.maximum(b, lax.rev(vs, (0,)))
  ms = _sortd(m)
  x = jnp.maximum(a, lax.rev(ms, (0,)))
  y = jnp.minimum(a, lax.rev(ms, (0,)))
  return _sortd(x), _sortd(y)


def _any16(mask):
  cnt = plsc.all_reduce_population_count(mask)
  return cnt[0] > 0


def _splat_lane(x, lane):
  idx = jnp.full((LANES, 1), lane, jnp.int32)
  dnums = lax.GatherDimensionNumbers(
      offset_dims=(), collapsed_slice_dims=(0,), start_index_map=(0,))
  return lax.gather(x, idx, dnums, (1,),
                    mode=lax.GatherScatterMode.PROMISE_IN_BOUNDS)


def _row_topk(buf):
  def maybe_merge(v, carry):
    a, b, t = carry

    def do(_):
      a2, b2 = _merge32(a, b, v)
      t2 = _splat_lane(b2, LANES - 1)
      return a2, b2, t2

    return lax.cond(_any16(v > t), do, lambda _: (a, b, t), None)

  def group_body(g, carry):
    a, b, t = carry
    base = g * (GROUP * LANES)
    vecs = [buf[pl.ds(base + j * LANES, LANES)] for j in range(GROUP)]
    gmax = vecs[0]
    for w in vecs[1:]:
      gmax = jnp.maximum(gmax, w)

    def rescan(carry):
      def inner(j, c):
        v = buf[pl.ds(base + j * LANES, LANES)]
        return maybe_merge(v, c)
      return lax.fori_loop(0, GROUP, inner, carry)

    return lax.cond(_any16(gmax > t), rescan, lambda c: c, carry)

  neg = jnp.full((LANES,), _NEG, jnp.float32)
  a, b, _ = lax.fori_loop(0, NGROUP, group_body, (neg, neg, neg))
  return a, b


def _sc_topk(logits):
  mesh = plsc.VectorSubcoreMesh(
      core_axis_name="c", subcore_axis_name="s", num_cores=NC, num_subcores=NS)

  @functools.partial(
      pl.kernel,
      out_type=jax.ShapeDtypeStruct((B, TOPK), jnp.float32),
      mesh=mesh,
      scratch_types=[
          pltpu.VMEM((V,), jnp.float32),
          pltpu.VMEM((TOPK,), jnp.float32),
      ],
      compiler_params=pltpu.CompilerParams(needs_layout_passes=False),
  )
  def k(logits_hbm, out_hbm, buf, obuf):
    wid = lax.axis_index("s") * NC + lax.axis_index("c")

    def row_body(r, _):
      row = wid * ROWS_PER_W + r
      pltpu.sync_copy(logits_hbm.at[row], buf)
      a, b = _row_topk(buf)
      obuf[pl.ds(0, LANES)] = a
      obuf[pl.ds(LANES, LANES)] = b
      pltpu.sync_copy(obuf, out_hbm.at[row])
      return 0

    lax.fori_loop(0, ROWS_PER_W, row_body, 0)

  return k(logits)


def _tc_head(x, ln_w, ln_b, w1, b1, w2, b2):
  def body(x_ref, lnw_ref, lnb_ref, w1_ref, b1_ref, w2_ref, b2_ref, o_ref):
    xv = x_ref[...]
    mean = jnp.mean(xv, axis=-1, keepdims=True)
    var = jnp.mean((xv - mean) ** 2, axis=-1, keepdims=True)
    xn = (xv - mean) * lax.rsqrt(var + 1e-5) * lnw_ref[...] + lnb_ref[...]
    h = jnp.dot(xn, w1_ref[...], preferred_element_type=jnp.float32) + b1_ref[...]
    h = 0.5 * h * (1.0 + lax.erf(h * jnp.float32(0.7071067811865476)))
    o_ref[...] = jnp.dot(h, w2_ref[...], preferred_element_type=jnp.float32) + b2_ref[...]

  return pl.pallas_call(
      body,
      out_shape=jax.ShapeDtypeStruct((B, OUT), jnp.float32),
  )(x, ln_w.reshape(1, TOPK), ln_b.reshape(1, TOPK),
    w1, b1.reshape(1, HID), w2, b2.reshape(1, OUT))


def kernel(logits, ln_w, ln_b, W1, b1, W2, b2):
  topk = _sc_topk(logits)
  return _tc_head(topk, ln_w, ln_b, W1, b1, W2, b2)
